# baseline (speedup 1.0000x reference)
import jax
import jax.numpy as jnp
from jax.experimental import pallas as pl
from jax.experimental.pallas import tpu as pltpu

N = 4096
D_IN = 512
D1 = 512
D2 = 256
DZ = 64

BM1 = 4096
BM = 512
NB = N // BM


def _s1_body(x_ref, w1_ref, s1_ref):
    xb = x_ref[...].astype(jnp.bfloat16)
    s1_ref[...] = jnp.dot(
        xb, w1_ref[...], preferred_element_type=jnp.float32
    ).astype(jnp.bfloat16)


def _prop_body(adj_ref, s1_ref, w2_ref, wzt_ref, bz_ref,
               h1_ref, h2_ref, z_ref, adjbf_ref, s2_ref):
    p = pl.program_id(0)
    m = pl.program_id(1)

    @pl.when(p == 0)
    def _phase0():
        ab = adj_ref[...].astype(jnp.bfloat16)
        adjbf_ref[pl.ds(m * BM, BM), :] = ab
        h1 = jnp.maximum(
            jnp.dot(ab, s1_ref[...], preferred_element_type=jnp.float32), 0.0
        )
        h1_ref[...] = h1
        s2_ref[pl.ds(m * BM, BM), :] = jnp.dot(
            h1.astype(jnp.bfloat16), w2_ref[...],
            preferred_element_type=jnp.float32,
        ).astype(jnp.bfloat16)

    @pl.when(p == 1)
    def _phase1():
        ab = adjbf_ref[pl.ds(m * BM, BM), :]
        h2 = jnp.maximum(
            jnp.dot(ab, s2_ref[...], preferred_element_type=jnp.float32), 0.0
        )
        h2_ref[...] = h2.astype(jnp.bfloat16)
        z_ref[...] = (
            jnp.dot(h2.astype(jnp.bfloat16), wzt_ref[...],
                    preferred_element_type=jnp.float32)
            + bz_ref[...]
        )


@jax.jit
def kernel(x, adj, W1, W2, Wz, bz):
    w1 = W1.astype(jnp.bfloat16)
    w2 = W2.astype(jnp.bfloat16)
    wzt = Wz.T.astype(jnp.bfloat16)
    bz2 = bz.reshape(1, DZ)

    s1 = pl.pallas_call(
        _s1_body,
        grid=(N // BM1,),
        in_specs=[
            pl.BlockSpec((BM1, D_IN), lambda m: (m, 0)),
            pl.BlockSpec((D_IN, D1), lambda m: (0, 0)),
        ],
        out_specs=pl.BlockSpec((BM1, D1), lambda m: (m, 0)),
        out_shape=jax.ShapeDtypeStruct((N, D1), jnp.bfloat16),
        compiler_params=pltpu.CompilerParams(
            dimension_semantics=("arbitrary",),
        ),
    )(x, w1)

    h1, h2, z = pl.pallas_call(
        _prop_body,
        grid=(2, NB),
        in_specs=[
            pl.BlockSpec((BM, N), lambda p, m: (jnp.where(p == 0, m, NB - 1), 0)),
            pl.BlockSpec((N, D1), lambda p, m: (0, 0)),
            pl.BlockSpec((D1, D2), lambda p, m: (0, 0)),
            pl.BlockSpec((D2, DZ), lambda p, m: (0, 0)),
            pl.BlockSpec((1, DZ), lambda p, m: (0, 0)),
        ],
        out_specs=[
            pl.BlockSpec((BM, D1), lambda p, m: (jnp.where(p == 0, m, NB - 1), 0)),
            pl.BlockSpec((BM, D2), lambda p, m: (jnp.where(p == 0, 0, m), 0)),
            pl.BlockSpec((BM, DZ), lambda p, m: (jnp.where(p == 0, 0, m), 0)),
        ],
        out_shape=[
            jax.ShapeDtypeStruct((N, D1), jnp.float32),
            jax.ShapeDtypeStruct((N, D2), jnp.bfloat16),
            jax.ShapeDtypeStruct((N, DZ), jnp.float32),
        ],
        scratch_shapes=[
            pltpu.VMEM((N, N), jnp.bfloat16),
            pltpu.VMEM((N, D2), jnp.bfloat16),
        ],
        compiler_params=pltpu.CompilerParams(
            dimension_semantics=("arbitrary", "arbitrary"),
        ),
    )(adj, s1, w2, wzt, bz2)

    return h1, h2.astype(jnp.float32), z


# speedup vs baseline: 1.0581x; 1.0581x over previous
import jax
import jax.numpy as jnp
from jax.experimental import pallas as pl
from jax.experimental.pallas import tpu as pltpu

N = 4096
D_IN = 512
D1 = 512
D2 = 256
DZ = 64

BM1 = 2048
BM = 512
NB = N // BM


def _s1_body(x_ref, w1_ref, s1_ref):
    xb = x_ref[...].astype(jnp.bfloat16)
    s1_ref[...] = jnp.dot(
        xb, w1_ref[...], preferred_element_type=jnp.float32
    ).astype(jnp.bfloat16)


def _prop_body(adj_ref, s1_ref, w2_ref, wzt_ref, bz_ref,
               h1_ref, h2_ref, z_ref, adjbf_ref, s2_ref):
    p = pl.program_id(0)
    m = pl.program_id(1)

    @pl.when(p == 0)
    def _phase0():
        ab = adj_ref[...].astype(jnp.bfloat16)
        adjbf_ref[pl.ds(m * BM, BM), :] = ab
        h1 = jnp.maximum(
            jnp.dot(ab, s1_ref[...], preferred_element_type=jnp.float32), 0.0
        )
        h1_ref[...] = h1
        s2_ref[pl.ds(m * BM, BM), :] = jnp.dot(
            h1.astype(jnp.bfloat16), w2_ref[...],
            preferred_element_type=jnp.float32,
        ).astype(jnp.bfloat16)

    @pl.when(p == 1)
    def _phase1():
        ab = adjbf_ref[pl.ds(m * BM, BM), :]
        h2 = jnp.maximum(
            jnp.dot(ab, s2_ref[...], preferred_element_type=jnp.float32), 0.0
        )
        h2_ref[...] = h2
        z_ref[...] = (
            jnp.dot(h2.astype(jnp.bfloat16), wzt_ref[...],
                    preferred_element_type=jnp.float32)
            + bz_ref[...]
        )


@jax.jit
def kernel(x, adj, W1, W2, Wz, bz):
    w1 = W1.astype(jnp.bfloat16)
    w2 = W2.astype(jnp.bfloat16)
    wzt = Wz.T.astype(jnp.bfloat16)
    bz2 = bz.reshape(1, DZ)

    s1 = pl.pallas_call(
        _s1_body,
        grid=(N // BM1,),
        in_specs=[
            pl.BlockSpec((BM1, D_IN), lambda m: (m, 0)),
            pl.BlockSpec((D_IN, D1), lambda m: (0, 0)),
        ],
        out_specs=pl.BlockSpec((BM1, D1), lambda m: (m, 0)),
        out_shape=jax.ShapeDtypeStruct((N, D1), jnp.bfloat16),
        compiler_params=pltpu.CompilerParams(
            dimension_semantics=("arbitrary",),
        ),
    )(x, w1)

    h1, h2, z = pl.pallas_call(
        _prop_body,
        grid=(2, NB),
        in_specs=[
            pl.BlockSpec((BM, N), lambda p, m: (jnp.where(p == 0, m, NB - 1), 0)),
            pl.BlockSpec((N, D1), lambda p, m: (0, 0)),
            pl.BlockSpec((D1, D2), lambda p, m: (0, 0)),
            pl.BlockSpec((D2, DZ), lambda p, m: (0, 0)),
            pl.BlockSpec((1, DZ), lambda p, m: (0, 0)),
        ],
        out_specs=[
            pl.BlockSpec((BM, D1), lambda p, m: (jnp.where(p == 0, m, NB - 1), 0)),
            pl.BlockSpec((BM, D2), lambda p, m: (jnp.where(p == 0, 0, m), 0)),
            pl.BlockSpec((BM, DZ), lambda p, m: (jnp.where(p == 0, 0, m), 0)),
        ],
        out_shape=[
            jax.ShapeDtypeStruct((N, D1), jnp.float32),
            jax.ShapeDtypeStruct((N, D2), jnp.float32),
            jax.ShapeDtypeStruct((N, DZ), jnp.float32),
        ],
        scratch_shapes=[
            pltpu.VMEM((N, N), jnp.bfloat16),
            pltpu.VMEM((N, D2), jnp.bfloat16),
        ],
        compiler_params=pltpu.CompilerParams(
            dimension_semantics=("arbitrary", "arbitrary"),
            vmem_limit_bytes=66584576,
        ),
    )(adj, s1, w2, wzt, bz2)

    return h1, h2, z
